# baseline (speedup 1.0000x reference)
import functools

import jax
import jax.numpy as jnp
from jax import lax
from jax.experimental import pallas as pl
from jax.experimental.pallas import tpu as pltpu
from jax.experimental.pallas import tpu_sc as plsc

N = 10000
E = 320000
H = 128
C = 10
L = 4

NC = 2
NS = 16
NW = NC * NS
CH = 128
NB = 2
EPAD = ((E + NW * CH * NB - 1) // (NW * CH * NB)) * (NW * CH * NB)
RPT = EPAD // (NW * CH)
ZB = 640
AGG_ROWS = NS * ZB

BLK = 1000


def _sc_scatter_body(um_hbm, src_hbm, dst_hbm, out_hbm, src_v, dst_v,
                     rows_a, agg_s, sem):
    cid = lax.axis_index("c")
    sid = lax.axis_index("s")
    wid = cid * NS + sid
    buf = lambda b: rows_a

    pltpu.sync_copy(src_hbm.at[wid], src_v)
    pltpu.sync_copy(dst_hbm.at[wid], dst_v)

    def _zero_row(i, _):
        z = jnp.zeros((16,), jnp.float32)
        for j in range(H // 16):
            rows_a[i, pl.ds(j * 16, 16)] = z
        return 0

    lax.fori_loop(0, CH, _zero_row, 0)
    for k in range(ZB // CH):
        pltpu.sync_copy(buf(0), agg_s.at[pl.ds(sid * ZB + k * CH, CH)])
    plsc.subcore_barrier()

    def _outer(g, _):
        j0 = g * NB
        for b in range(NB):
            j = j0 + b
            pltpu.async_copy(um_hbm.at[src_v.at[j]], buf(b), sem).wait()
            pltpu.sync_copy(buf(b), agg_s.at[dst_v.at[j]], add=True)
        return 0

    lax.fori_loop(0, RPT // NB, _outer, 0)
    plsc.subcore_barrier()

    for k in range(ZB // CH):
        sl = pl.ds(sid * ZB + k * CH, CH)
        pltpu.sync_copy(agg_s.at[sl], buf(0))
        pltpu.sync_copy(buf(0), out_hbm.at[cid].at[sl])


@functools.cache
def _sc_scatter_build():
    return pl.kernel(
        _sc_scatter_body,
        out_type=jax.ShapeDtypeStruct((NC, AGG_ROWS, H), jnp.float32),
        mesh=plsc.VectorSubcoreMesh(core_axis_name="c", subcore_axis_name="s",
                                    num_cores=NC, num_subcores=NS),
        scratch_types=[
            pltpu.VMEM((RPT, CH), jnp.int32),
            pltpu.VMEM((RPT, CH), jnp.int32),
            pltpu.VMEM((CH, H), jnp.float32),
            pltpu.VMEM_SHARED((AGG_ROWS, H), jnp.float32),
            pltpu.SemaphoreType.DMA,
        ],
    )


def _sc_scatter(um, src_p, dst_p):
    return _sc_scatter_build()(um, src_p, dst_p)


def _tc_init_body(x_ref, wi_ref, bi_ref, wn_ref, bn_ref, wn1_ref, bn1_ref,
                  wn2_ref, bn2_ref, u_ref, g_ref, acc_ref):
    i = pl.program_id(0)

    @pl.when(i == 0)
    def _():
        acc_ref[...] = jnp.zeros_like(acc_ref)

    xb = x_ref[...]
    u_ref[...] = (
        jnp.dot(xb, wi_ref[...], preferred_element_type=jnp.float32) + bi_ref[...]
    )
    acc_ref[...] += jnp.sum(xb, axis=0, keepdims=True)

    @pl.when(i == pl.num_programs(0) - 1)
    def _():
        m = acc_ref[...] * (1.0 / N)
        g = jnp.dot(m, wn_ref[...], preferred_element_type=jnp.float32) + bn_ref[...]
        h = jnp.maximum(
            jnp.dot(g, wn1_ref[...], preferred_element_type=jnp.float32) + bn1_ref[...],
            0.0,
        )
        g_ref[...] = (
            g + jnp.dot(h, wn2_ref[...], preferred_element_type=jnp.float32) + bn2_ref[...]
        )


def _tc_init(x, W_init, b_init, W_np, b_np, W_np1, b_np1, W_np2, b_np2):
    full = lambda: pl.BlockSpec((H, H), lambda i: (0, 0))
    vec = lambda: pl.BlockSpec((1, H), lambda i: (0, 0))
    return pl.pallas_call(
        _tc_init_body,
        grid=(N // BLK,),
        in_specs=[
            pl.BlockSpec((BLK, H), lambda i: (i, 0)),
            full(), vec(), full(), vec(), full(), vec(), full(), vec(),
        ],
        out_specs=[
            pl.BlockSpec((BLK, H), lambda i: (i, 0)),
            pl.BlockSpec((1, H), lambda i: (0, 0)),
        ],
        out_shape=[
            jax.ShapeDtypeStruct((N, H), jnp.float32),
            jax.ShapeDtypeStruct((1, H), jnp.float32),
        ],
        scratch_shapes=[pltpu.VMEM((1, H), jnp.float32)],
    )(x, W_init, b_init, W_np, b_np, W_np1, b_np1, W_np2, b_np2)


def _tc_pre_body(u_ref, s_ref, t_ref, wm_ref, bm_ref, um_ref):
    un = u_ref[...] * s_ref[...] + t_ref[...]
    um_ref[...] = (
        jnp.dot(un, wm_ref[...], preferred_element_type=jnp.float32) + bm_ref[...]
    )


def _tc_pre(u, s, t, Wm_i, bm_i):
    return pl.pallas_call(
        _tc_pre_body,
        grid=(N // BLK,),
        in_specs=[
            pl.BlockSpec((BLK, H), lambda i: (i, 0)),
            pl.BlockSpec((1, H), lambda i: (0, 0)),
            pl.BlockSpec((1, H), lambda i: (0, 0)),
            pl.BlockSpec((H, H), lambda i: (0, 0)),
            pl.BlockSpec((1, H), lambda i: (0, 0)),
        ],
        out_specs=pl.BlockSpec((BLK, H), lambda i: (i, 0)),
        out_shape=jax.ShapeDtypeStruct((N, H), jnp.float32),
    )(u, s, t, Wm_i, bm_i)


def _tc_post_body(p0_ref, p1_ref, um_ref, wi_ref, bi_ref, wj_ref, bj_ref,
                  gam_ref, bet_ref, we_ref, be_ref, we1_ref, be1_ref,
                  we2_ref, be2_ref, u_ref, s_ref, t_ref, ge_ref,
                  accs_ref, accq_ref):
    i = pl.program_id(0)

    @pl.when(i == 0)
    def _():
        accs_ref[...] = jnp.zeros_like(accs_ref)
        accq_ref[...] = jnp.zeros_like(accq_ref)

    agg = (p0_ref[0] + p1_ref[0]) * (float(N) / float(E))
    um = um_ref[...]
    ai = wi_ref[...] * um + bi_ref[...]
    aj = wj_ref[...] * agg + bj_ref[...]
    u = agg + um + ai * aj
    u_ref[...] = u
    accs_ref[...] += jnp.sum(u, axis=0, keepdims=True)
    accq_ref[...] += jnp.sum(u * u, axis=0, keepdims=True)

    @pl.when(i == pl.num_programs(0) - 1)
    def _():
        mu = accs_ref[...] * (1.0 / N)
        var = accq_ref[...] * (1.0 / N) - mu * mu
        s = gam_ref[...] * lax.rsqrt(var + 1e-5)
        s_ref[...] = s
        t_ref[...] = bet_ref[...] - mu * s
        ge = jnp.dot(mu, we_ref[...], preferred_element_type=jnp.float32) + be_ref[...]
        h = jnp.maximum(
            jnp.dot(ge, we1_ref[...], preferred_element_type=jnp.float32) + be1_ref[...],
            0.0,
        )
        ge_ref[...] = (
            ge + jnp.dot(h, we2_ref[...], preferred_element_type=jnp.float32) + be2_ref[...]
        )


def _tc_post(parts, um, wi_i, bi_i, wj_i, bj_i, gam_n, bet_n,
             We_i, be_i, We1_i, be1_i, We2_i, be2_i):
    full = lambda: pl.BlockSpec((H, H), lambda i: (0, 0))
    vec = lambda: pl.BlockSpec((1, H), lambda i: (0, 0))
    return pl.pallas_call(
        _tc_post_body,
        grid=(N // BLK,),
        in_specs=[
            pl.BlockSpec((1, BLK, H), lambda i: (0, i, 0)),
            pl.BlockSpec((1, BLK, H), lambda i: (1, i, 0)),
            pl.BlockSpec((BLK, H), lambda i: (i, 0)),
            vec(), vec(), vec(), vec(), vec(), vec(),
            full(), vec(), full(), vec(), full(), vec(),
        ],
        out_specs=[
            pl.BlockSpec((BLK, H), lambda i: (i, 0)),
            pl.BlockSpec((1, H), lambda i: (0, 0)),
            pl.BlockSpec((1, H), lambda i: (0, 0)),
            pl.BlockSpec((1, H), lambda i: (0, 0)),
        ],
        out_shape=[
            jax.ShapeDtypeStruct((N, H), jnp.float32),
            jax.ShapeDtypeStruct((1, H), jnp.float32),
            jax.ShapeDtypeStruct((1, H), jnp.float32),
            jax.ShapeDtypeStruct((1, H), jnp.float32),
        ],
        scratch_shapes=[
            pltpu.VMEM((1, H), jnp.float32),
            pltpu.VMEM((1, H), jnp.float32),
        ],
    )(parts, parts, um, wi_i, bi_i, wj_i, bj_i, gam_n, bet_n,
      We_i, be_i, We1_i, be1_i, We2_i, be2_i)


def _tc_final_body(g_ref, ge0_ref, ge1_ref, ge2_ref, ge3_ref, wac_ref, bac_ref,
                   wf_ref, bf_ref, out_ref):
    out = g_ref[...] + (ge0_ref[...] + ge1_ref[...] + ge2_ref[...] + ge3_ref[...]) * (1.0 / L)
    h = jnp.maximum(
        jnp.dot(out, wac_ref[...], preferred_element_type=jnp.float32) + bac_ref[...],
        0.0,
    )
    out = h + out
    logits = jnp.dot(out, wf_ref[...], preferred_element_type=jnp.float32) + bf_ref[...]
    m = jnp.max(logits, axis=-1, keepdims=True)
    lse = jnp.log(jnp.sum(jnp.exp(logits - m), axis=-1, keepdims=True)) + m
    out_ref[...] = logits - lse


def _tc_final(g, ge0, ge1, ge2, ge3, W_ac, b_ac, W_f_pad, b_f_pad):
    full = lambda: pl.BlockSpec((H, H), lambda: (0, 0))
    vec = lambda: pl.BlockSpec((1, H), lambda: (0, 0))
    return pl.pallas_call(
        _tc_final_body,
        grid=(),
        in_specs=[vec(), vec(), vec(), vec(), vec(), full(), vec(), full(), vec()],
        out_specs=pl.BlockSpec((1, H), lambda: (0, 0)),
        out_shape=jax.ShapeDtypeStruct((1, H), jnp.float32),
    )(g, ge0, ge1, ge2, ge3, W_ac, b_ac, W_f_pad, b_f_pad)


def kernel(x, edge_index, W_np, b_np, W_np1, b_np1, W_np2, b_np2, W_init, b_init,
           Wm, bm, wi, bi, wj, bj, gamma, beta, We, be, We1, be1, We2, be2,
           W_ac, b_ac, W_f, b_f):
    r = lambda v: v.reshape(1, -1)

    src = edge_index[0]
    dst = edge_index[1]
    pad = EPAD - E
    trash = N + jnp.arange(pad, dtype=jnp.int32) % (AGG_ROWS - N)
    src_p = jnp.concatenate([src, jnp.zeros((pad,), jnp.int32)]).reshape(NW, RPT, CH)
    dst_p = jnp.concatenate([dst, trash]).reshape(NW, RPT, CH)

    u, g = _tc_init(x, W_init, r(b_init), W_np, r(b_np), W_np1, r(b_np1),
                    W_np2, r(b_np2))

    s = jnp.ones((1, H), jnp.float32)
    t = jnp.zeros((1, H), jnp.float32)
    ges = []
    for i in range(L):
        um = _tc_pre(u, s, t, Wm[i], r(bm[i]))
        parts = _sc_scatter(um, src_p, dst_p)
        u, s, t, ge = _tc_post(
            parts, um, r(wi[i]), r(bi[i]), r(wj[i]), r(bj[i]),
            r(gamma[(i + 1) % L]), r(beta[(i + 1) % L]),
            We[i], r(be[i]), We1[i], r(be1[i]), We2[i], r(be2[i]))
        ges.append(ge)

    W_f_pad = jnp.zeros((H, H), jnp.float32).at[:, :C].set(W_f)
    b_f_pad = jnp.full((1, H), -1e30, jnp.float32).at[:, :C].set(b_f)
    out = _tc_final(g, ges[0], ges[1], ges[2], ges[3], W_ac, r(b_ac), W_f_pad, b_f_pad)
    return out[:, :C]


# speedup vs baseline: 2.4568x; 2.4568x over previous
import functools

import jax
import jax.numpy as jnp
from jax import lax
from jax.experimental import pallas as pl
from jax.experimental.pallas import tpu as pltpu
from jax.experimental.pallas import tpu_sc as plsc

N = 10000
E = 320000
H = 128
C = 10
L = 4

NC = 2
NS = 16
NW = NC * NS
CH = 128
NB = 2
EPAD = ((E + NW * CH * NB - 1) // (NW * CH * NB)) * (NW * CH * NB)
RPT = EPAD // (NW * CH)
ZB = 640
AGG_ROWS = NS * ZB

BLK = 1000


def _sc_scatter_body(um_hbm, src_hbm, dst_hbm, out_hbm, src_v, dst_v,
                     rows_a, agg_s, sem):
    cid = lax.axis_index("c")
    sid = lax.axis_index("s")
    wid = cid * NS + sid
    buf = lambda b: rows_a

    pltpu.sync_copy(src_hbm.at[wid], src_v)
    pltpu.sync_copy(dst_hbm.at[wid], dst_v)

    def _zero_row(i, _):
        z = jnp.zeros((16,), jnp.float32)
        for j in range(H // 16):
            rows_a[i, pl.ds(j * 16, 16)] = z
        return 0

    lax.fori_loop(0, CH, _zero_row, 0)
    for k in range(ZB // CH):
        pltpu.sync_copy(buf(0), agg_s.at[pl.ds(sid * ZB + k * CH, CH)])
    plsc.subcore_barrier()

    def _outer(g, _):
        j0 = g * NB
        for b in range(NB):
            j = j0 + b
            pltpu.async_copy(um_hbm.at[src_v.at[j]], buf(b), sem).wait()
            pltpu.sync_copy(buf(b), agg_s.at[dst_v.at[j]], add=True)
        return 0

    lax.fori_loop(0, RPT // NB, _outer, 0)
    plsc.subcore_barrier()

    for k in range(ZB // CH):
        sl = pl.ds(sid * ZB + k * CH, CH)
        pltpu.sync_copy(agg_s.at[sl], buf(0))
        pltpu.sync_copy(buf(0), out_hbm.at[cid].at[sl])


@functools.cache
def _sc_scatter_build():
    return pl.kernel(
        _sc_scatter_body,
        out_type=jax.ShapeDtypeStruct((NC, AGG_ROWS, H), jnp.float32),
        mesh=plsc.VectorSubcoreMesh(core_axis_name="c", subcore_axis_name="s",
                                    num_cores=NC, num_subcores=NS),
        scratch_types=[
            pltpu.VMEM((RPT, CH), jnp.int32),
            pltpu.VMEM((RPT, CH), jnp.int32),
            pltpu.VMEM((CH, H), jnp.float32),
            pltpu.VMEM_SHARED((AGG_ROWS, H), jnp.float32),
            pltpu.SemaphoreType.DMA,
        ],
    )


def _sc_scatter(um, src_p, dst_p):
    return _sc_scatter_build()(um, src_p, dst_p)


def _tc_init_body(x_ref, wi_ref, bi_ref, wn_ref, bn_ref, wn1_ref, bn1_ref,
                  wn2_ref, bn2_ref, u_ref, g_ref, acc_ref):
    i = pl.program_id(0)

    @pl.when(i == 0)
    def _():
        acc_ref[...] = jnp.zeros_like(acc_ref)

    xb = x_ref[...]
    u_ref[...] = (
        jnp.dot(xb, wi_ref[...], preferred_element_type=jnp.float32) + bi_ref[...]
    )
    acc_ref[...] += jnp.sum(xb, axis=0, keepdims=True)

    @pl.when(i == pl.num_programs(0) - 1)
    def _():
        m = acc_ref[...] * (1.0 / N)
        g = jnp.dot(m, wn_ref[...], preferred_element_type=jnp.float32) + bn_ref[...]
        h = jnp.maximum(
            jnp.dot(g, wn1_ref[...], preferred_element_type=jnp.float32) + bn1_ref[...],
            0.0,
        )
        g_ref[...] = (
            g + jnp.dot(h, wn2_ref[...], preferred_element_type=jnp.float32) + bn2_ref[...]
        )


def _tc_init(x, W_init, b_init, W_np, b_np, W_np1, b_np1, W_np2, b_np2):
    full = lambda: pl.BlockSpec((H, H), lambda i: (0, 0))
    vec = lambda: pl.BlockSpec((1, H), lambda i: (0, 0))
    return pl.pallas_call(
        _tc_init_body,
        grid=(N // BLK,),
        in_specs=[
            pl.BlockSpec((BLK, H), lambda i: (i, 0)),
            full(), vec(), full(), vec(), full(), vec(), full(), vec(),
        ],
        out_specs=[
            pl.BlockSpec((BLK, H), lambda i: (i, 0)),
            pl.BlockSpec((1, H), lambda i: (0, 0)),
        ],
        out_shape=[
            jax.ShapeDtypeStruct((N, H), jnp.float32),
            jax.ShapeDtypeStruct((1, H), jnp.float32),
        ],
        scratch_shapes=[pltpu.VMEM((1, H), jnp.float32)],
    )(x, W_init, b_init, W_np, b_np, W_np1, b_np1, W_np2, b_np2)


def _tc_pre_body(u_ref, s_ref, t_ref, wm_ref, bm_ref, um_ref):
    un = u_ref[...] * s_ref[...] + t_ref[...]
    um_ref[...] = (
        jnp.dot(un, wm_ref[...], preferred_element_type=jnp.float32) + bm_ref[...]
    )


def _tc_pre(u, s, t, Wm_i, bm_i):
    return pl.pallas_call(
        _tc_pre_body,
        grid=(N // BLK,),
        in_specs=[
            pl.BlockSpec((BLK, H), lambda i: (i, 0)),
            pl.BlockSpec((1, H), lambda i: (0, 0)),
            pl.BlockSpec((1, H), lambda i: (0, 0)),
            pl.BlockSpec((H, H), lambda i: (0, 0)),
            pl.BlockSpec((1, H), lambda i: (0, 0)),
        ],
        out_specs=pl.BlockSpec((BLK, H), lambda i: (i, 0)),
        out_shape=jax.ShapeDtypeStruct((N, H), jnp.float32),
    )(u, s, t, Wm_i, bm_i)


def _tc_post_body(p0_ref, p1_ref, um_ref, wi_ref, bi_ref, wj_ref, bj_ref,
                  gam_ref, bet_ref, we_ref, be_ref, we1_ref, be1_ref,
                  we2_ref, be2_ref, u_ref, s_ref, t_ref, ge_ref,
                  accs_ref, accq_ref):
    i = pl.program_id(0)

    @pl.when(i == 0)
    def _():
        accs_ref[...] = jnp.zeros_like(accs_ref)
        accq_ref[...] = jnp.zeros_like(accq_ref)

    agg = (p0_ref[0] + p1_ref[0]) * (float(N) / float(E))
    um = um_ref[...]
    ai = wi_ref[...] * um + bi_ref[...]
    aj = wj_ref[...] * agg + bj_ref[...]
    u = agg + um + ai * aj
    u_ref[...] = u
    accs_ref[...] += jnp.sum(u, axis=0, keepdims=True)
    accq_ref[...] += jnp.sum(u * u, axis=0, keepdims=True)

    @pl.when(i == pl.num_programs(0) - 1)
    def _():
        mu = accs_ref[...] * (1.0 / N)
        var = accq_ref[...] * (1.0 / N) - mu * mu
        s = gam_ref[...] * lax.rsqrt(var + 1e-5)
        s_ref[...] = s
        t_ref[...] = bet_ref[...] - mu * s
        ge = jnp.dot(mu, we_ref[...], preferred_element_type=jnp.float32) + be_ref[...]
        h = jnp.maximum(
            jnp.dot(ge, we1_ref[...], preferred_element_type=jnp.float32) + be1_ref[...],
            0.0,
        )
        ge_ref[...] = (
            ge + jnp.dot(h, we2_ref[...], preferred_element_type=jnp.float32) + be2_ref[...]
        )


def _tc_post(parts, um, wi_i, bi_i, wj_i, bj_i, gam_n, bet_n,
             We_i, be_i, We1_i, be1_i, We2_i, be2_i):
    full = lambda: pl.BlockSpec((H, H), lambda i: (0, 0))
    vec = lambda: pl.BlockSpec((1, H), lambda i: (0, 0))
    return pl.pallas_call(
        _tc_post_body,
        grid=(N // BLK,),
        in_specs=[
            pl.BlockSpec((1, BLK, H), lambda i: (0, i, 0)),
            pl.BlockSpec((1, BLK, H), lambda i: (1, i, 0)),
            pl.BlockSpec((BLK, H), lambda i: (i, 0)),
            vec(), vec(), vec(), vec(), vec(), vec(),
            full(), vec(), full(), vec(), full(), vec(),
        ],
        out_specs=[
            pl.BlockSpec((BLK, H), lambda i: (i, 0)),
            pl.BlockSpec((1, H), lambda i: (0, 0)),
            pl.BlockSpec((1, H), lambda i: (0, 0)),
            pl.BlockSpec((1, H), lambda i: (0, 0)),
        ],
        out_shape=[
            jax.ShapeDtypeStruct((N, H), jnp.float32),
            jax.ShapeDtypeStruct((1, H), jnp.float32),
            jax.ShapeDtypeStruct((1, H), jnp.float32),
            jax.ShapeDtypeStruct((1, H), jnp.float32),
        ],
        scratch_shapes=[
            pltpu.VMEM((1, H), jnp.float32),
            pltpu.VMEM((1, H), jnp.float32),
        ],
    )(parts, parts, um, wi_i, bi_i, wj_i, bj_i, gam_n, bet_n,
      We_i, be_i, We1_i, be1_i, We2_i, be2_i)


def _tc_final_body(g_ref, ge0_ref, ge1_ref, ge2_ref, ge3_ref, wac_ref, bac_ref,
                   wf_ref, bf_ref, out_ref):
    out = g_ref[...] + (ge0_ref[...] + ge1_ref[...] + ge2_ref[...] + ge3_ref[...]) * (1.0 / L)
    h = jnp.maximum(
        jnp.dot(out, wac_ref[...], preferred_element_type=jnp.float32) + bac_ref[...],
        0.0,
    )
    out = h + out
    logits = jnp.dot(out, wf_ref[...], preferred_element_type=jnp.float32) + bf_ref[...]
    m = jnp.max(logits, axis=-1, keepdims=True)
    lse = jnp.log(jnp.sum(jnp.exp(logits - m), axis=-1, keepdims=True)) + m
    out_ref[...] = logits - lse


def _tc_final(g, ge0, ge1, ge2, ge3, W_ac, b_ac, W_f_pad, b_f_pad):
    full = lambda: pl.BlockSpec((H, H), lambda: (0, 0))
    vec = lambda: pl.BlockSpec((1, H), lambda: (0, 0))
    return pl.pallas_call(
        _tc_final_body,
        grid=(),
        in_specs=[vec(), vec(), vec(), vec(), vec(), full(), vec(), full(), vec()],
        out_specs=pl.BlockSpec((1, H), lambda: (0, 0)),
        out_shape=jax.ShapeDtypeStruct((1, H), jnp.float32),
    )(g, ge0, ge1, ge2, ge3, W_ac, b_ac, W_f_pad, b_f_pad)


def kernel(x, edge_index, W_np, b_np, W_np1, b_np1, W_np2, b_np2, W_init, b_init,
           Wm, bm, wi, bi, wj, bj, gamma, beta, We, be, We1, be1, We2, be2,
           W_ac, b_ac, W_f, b_f):
    r = lambda v: v.reshape(1, -1)

    src = edge_index[0]
    dst = edge_index[1]
    pad = EPAD - E
    arp = jnp.arange(pad, dtype=jnp.int32)
    trash = N + arp % (AGG_ROWS - N)
    src_p = jnp.concatenate([src, arp % N]).reshape(NW, RPT, CH)
    dst_p = jnp.concatenate([dst, trash]).reshape(NW, RPT, CH)

    u, g = _tc_init(x, W_init, r(b_init), W_np, r(b_np), W_np1, r(b_np1),
                    W_np2, r(b_np2))

    s = jnp.ones((1, H), jnp.float32)
    t = jnp.zeros((1, H), jnp.float32)
    ges = []
    for i in range(L):
        um = _tc_pre(u, s, t, Wm[i], r(bm[i]))
        parts = _sc_scatter(um, src_p, dst_p)
        u, s, t, ge = _tc_post(
            parts, um, r(wi[i]), r(bi[i]), r(wj[i]), r(bj[i]),
            r(gamma[(i + 1) % L]), r(beta[(i + 1) % L]),
            We[i], r(be[i]), We1[i], r(be1[i]), We2[i], r(be2[i]))
        ges.append(ge)

    W_f_pad = jnp.zeros((H, H), jnp.float32).at[:, :C].set(W_f)
    b_f_pad = jnp.full((1, H), -1e30, jnp.float32).at[:, :C].set(b_f)
    out = _tc_final(g, ges[0], ges[1], ges[2], ges[3], W_ac, r(b_ac), W_f_pad, b_f_pad)
    return out[:, :C]


# speedup vs baseline: 3.7408x; 1.5226x over previous
import functools

import jax
import jax.numpy as jnp
from jax import lax
from jax.experimental import pallas as pl
from jax.experimental.pallas import tpu as pltpu
from jax.experimental.pallas import tpu_sc as plsc

N = 10000
E = 320000
H = 128
C = 10
L = 4

NC = 2
NS = 16
NW = NC * NS
CH = 128
ROWD = 2
IDXD = 4
EPAD = ((E + NW * CH * IDXD - 1) // (NW * CH * IDXD)) * (NW * CH * IDXD)
RPT = EPAD // (NW * CH)
ZB = 640
AGG_ROWS = NS * ZB

BLK = 1000


def _sc_scatter_body(um_hbm, src_hbm, dst_hbm, out_hbm, src_r, dst_r,
                     rows_a, rows_b, agg_s, sem_g, sem_i):
    cid = lax.axis_index("c")
    sid = lax.axis_index("s")
    wid = cid * NS + sid
    bufs = (rows_a, rows_b)

    def idx_load(j, slot):
        return (pltpu.make_async_copy(src_hbm.at[wid].at[j], src_r.at[slot], sem_i),
                pltpu.make_async_copy(dst_hbm.at[wid].at[j], dst_r.at[slot], sem_i))

    def gather(slot, b):
        return pltpu.make_async_copy(um_hbm.at[src_r.at[slot]], bufs[b], sem_g)

    def _zero_row(i, _):
        z = jnp.zeros((16,), jnp.float32)
        for j in range(H // 16):
            rows_a[i, pl.ds(j * 16, 16)] = z
        return 0

    lax.fori_loop(0, CH, _zero_row, 0)
    for k in range(ZB // CH):
        pltpu.sync_copy(rows_a, agg_s.at[pl.ds(sid * ZB + k * CH, CH)])
    plsc.subcore_barrier()

    for s in range(IDXD):
        d1, d2 = idx_load(s, s)
        d1.start(); d2.start()
    for s in range(ROWD):
        d1, d2 = idx_load(s, s)
        d1.wait(); d2.wait()
        gather(s, s).start()

    def _outer(g, _):
        j0 = g * IDXD
        for b in range(IDXD):
            j = j0 + b
            gather(b, b % ROWD).wait()
            pltpu.sync_copy(bufs[b % ROWD], agg_s.at[dst_r.at[b]], add=True)
            d1, d2 = idx_load(j + IDXD, b)
            d1.start(); d2.start()
            nslot = (b + ROWD) % IDXD
            w1, w2 = idx_load(j + ROWD, nslot)
            w1.wait(); w2.wait()
            gather(nslot, b % ROWD).start()
        return 0

    lax.fori_loop(0, RPT // IDXD - 1, _outer, 0)
    j0 = RPT - IDXD
    for b in range(IDXD):
        j = j0 + b
        gather(b, b % ROWD).wait()
        pltpu.sync_copy(bufs[b % ROWD], agg_s.at[dst_r.at[b]], add=True)
        if b + ROWD < IDXD:
            nslot = b + ROWD
            w1, w2 = idx_load(j + ROWD, nslot)
            w1.wait(); w2.wait()
            gather(nslot, b % ROWD).start()
    plsc.subcore_barrier()

    for k in range(ZB // CH):
        sl = pl.ds(sid * ZB + k * CH, CH)
        pltpu.sync_copy(agg_s.at[sl], rows_a)
        pltpu.sync_copy(rows_a, out_hbm.at[cid].at[sl])


@functools.cache
def _sc_scatter_build():
    return pl.kernel(
        _sc_scatter_body,
        out_type=jax.ShapeDtypeStruct((NC, AGG_ROWS, H), jnp.float32),
        mesh=plsc.VectorSubcoreMesh(core_axis_name="c", subcore_axis_name="s",
                                    num_cores=NC, num_subcores=NS),
        scratch_types=[
            pltpu.VMEM((IDXD, CH), jnp.int32),
            pltpu.VMEM((IDXD, CH), jnp.int32),
            pltpu.VMEM((CH, H), jnp.float32),
            pltpu.VMEM((CH, H), jnp.float32),
            pltpu.VMEM_SHARED((AGG_ROWS, H), jnp.float32),
            pltpu.SemaphoreType.DMA,
            pltpu.SemaphoreType.DMA,
        ],
    )


def _sc_scatter(um, src_p, dst_p):
    return _sc_scatter_build()(um, src_p, dst_p)


def _tc_init_body(x_ref, wi_ref, bi_ref, wn_ref, bn_ref, wn1_ref, bn1_ref,
                  wn2_ref, bn2_ref, u_ref, g_ref, acc_ref):
    i = pl.program_id(0)

    @pl.when(i == 0)
    def _():
        acc_ref[...] = jnp.zeros_like(acc_ref)

    xb = x_ref[...]
    u_ref[...] = (
        jnp.dot(xb, wi_ref[...], preferred_element_type=jnp.float32) + bi_ref[...]
    )
    acc_ref[...] += jnp.sum(xb, axis=0, keepdims=True)

    @pl.when(i == pl.num_programs(0) - 1)
    def _():
        m = acc_ref[...] * (1.0 / N)
        g = jnp.dot(m, wn_ref[...], preferred_element_type=jnp.float32) + bn_ref[...]
        h = jnp.maximum(
            jnp.dot(g, wn1_ref[...], preferred_element_type=jnp.float32) + bn1_ref[...],
            0.0,
        )
        g_ref[...] = (
            g + jnp.dot(h, wn2_ref[...], preferred_element_type=jnp.float32) + bn2_ref[...]
        )


def _tc_init(x, W_init, b_init, W_np, b_np, W_np1, b_np1, W_np2, b_np2):
    full = lambda: pl.BlockSpec((H, H), lambda i: (0, 0))
    vec = lambda: pl.BlockSpec((1, H), lambda i: (0, 0))
    return pl.pallas_call(
        _tc_init_body,
        grid=(N // BLK,),
        in_specs=[
            pl.BlockSpec((BLK, H), lambda i: (i, 0)),
            full(), vec(), full(), vec(), full(), vec(), full(), vec(),
        ],
        out_specs=[
            pl.BlockSpec((BLK, H), lambda i: (i, 0)),
            pl.BlockSpec((1, H), lambda i: (0, 0)),
        ],
        out_shape=[
            jax.ShapeDtypeStruct((N, H), jnp.float32),
            jax.ShapeDtypeStruct((1, H), jnp.float32),
        ],
        scratch_shapes=[pltpu.VMEM((1, H), jnp.float32)],
    )(x, W_init, b_init, W_np, b_np, W_np1, b_np1, W_np2, b_np2)


def _tc_pre_body(u_ref, s_ref, t_ref, wm_ref, bm_ref, um_ref):
    un = u_ref[...] * s_ref[...] + t_ref[...]
    um_ref[...] = (
        jnp.dot(un, wm_ref[...], preferred_element_type=jnp.float32) + bm_ref[...]
    )


def _tc_pre(u, s, t, Wm_i, bm_i):
    return pl.pallas_call(
        _tc_pre_body,
        grid=(N // BLK,),
        in_specs=[
            pl.BlockSpec((BLK, H), lambda i: (i, 0)),
            pl.BlockSpec((1, H), lambda i: (0, 0)),
            pl.BlockSpec((1, H), lambda i: (0, 0)),
            pl.BlockSpec((H, H), lambda i: (0, 0)),
            pl.BlockSpec((1, H), lambda i: (0, 0)),
        ],
        out_specs=pl.BlockSpec((BLK, H), lambda i: (i, 0)),
        out_shape=jax.ShapeDtypeStruct((N, H), jnp.float32),
    )(u, s, t, Wm_i, bm_i)


def _tc_post_body(p0_ref, p1_ref, um_ref, wi_ref, bi_ref, wj_ref, bj_ref,
                  gam_ref, bet_ref, we_ref, be_ref, we1_ref, be1_ref,
                  we2_ref, be2_ref, u_ref, s_ref, t_ref, ge_ref,
                  accs_ref, accq_ref):
    i = pl.program_id(0)

    @pl.when(i == 0)
    def _():
        accs_ref[...] = jnp.zeros_like(accs_ref)
        accq_ref[...] = jnp.zeros_like(accq_ref)

    agg = (p0_ref[0] + p1_ref[0]) * (float(N) / float(E))
    um = um_ref[...]
    ai = wi_ref[...] * um + bi_ref[...]
    aj = wj_ref[...] * agg + bj_ref[...]
    u = agg + um + ai * aj
    u_ref[...] = u
    accs_ref[...] += jnp.sum(u, axis=0, keepdims=True)
    accq_ref[...] += jnp.sum(u * u, axis=0, keepdims=True)

    @pl.when(i == pl.num_programs(0) - 1)
    def _():
        mu = accs_ref[...] * (1.0 / N)
        var = accq_ref[...] * (1.0 / N) - mu * mu
        s = gam_ref[...] * lax.rsqrt(var + 1e-5)
        s_ref[...] = s
        t_ref[...] = bet_ref[...] - mu * s
        ge = jnp.dot(mu, we_ref[...], preferred_element_type=jnp.float32) + be_ref[...]
        h = jnp.maximum(
            jnp.dot(ge, we1_ref[...], preferred_element_type=jnp.float32) + be1_ref[...],
            0.0,
        )
        ge_ref[...] = (
            ge + jnp.dot(h, we2_ref[...], preferred_element_type=jnp.float32) + be2_ref[...]
        )


def _tc_post(parts, um, wi_i, bi_i, wj_i, bj_i, gam_n, bet_n,
             We_i, be_i, We1_i, be1_i, We2_i, be2_i):
    full = lambda: pl.BlockSpec((H, H), lambda i: (0, 0))
    vec = lambda: pl.BlockSpec((1, H), lambda i: (0, 0))
    return pl.pallas_call(
        _tc_post_body,
        grid=(N // BLK,),
        in_specs=[
            pl.BlockSpec((1, BLK, H), lambda i: (0, i, 0)),
            pl.BlockSpec((1, BLK, H), lambda i: (1, i, 0)),
            pl.BlockSpec((BLK, H), lambda i: (i, 0)),
            vec(), vec(), vec(), vec(), vec(), vec(),
            full(), vec(), full(), vec(), full(), vec(),
        ],
        out_specs=[
            pl.BlockSpec((BLK, H), lambda i: (i, 0)),
            pl.BlockSpec((1, H), lambda i: (0, 0)),
            pl.BlockSpec((1, H), lambda i: (0, 0)),
            pl.BlockSpec((1, H), lambda i: (0, 0)),
        ],
        out_shape=[
            jax.ShapeDtypeStruct((N, H), jnp.float32),
            jax.ShapeDtypeStruct((1, H), jnp.float32),
            jax.ShapeDtypeStruct((1, H), jnp.float32),
            jax.ShapeDtypeStruct((1, H), jnp.float32),
        ],
        scratch_shapes=[
            pltpu.VMEM((1, H), jnp.float32),
            pltpu.VMEM((1, H), jnp.float32),
        ],
    )(parts, parts, um, wi_i, bi_i, wj_i, bj_i, gam_n, bet_n,
      We_i, be_i, We1_i, be1_i, We2_i, be2_i)


def _tc_final_body(g_ref, ge0_ref, ge1_ref, ge2_ref, ge3_ref, wac_ref, bac_ref,
                   wf_ref, bf_ref, out_ref):
    out = g_ref[...] + (ge0_ref[...] + ge1_ref[...] + ge2_ref[...] + ge3_ref[...]) * (1.0 / L)
    h = jnp.maximum(
        jnp.dot(out, wac_ref[...], preferred_element_type=jnp.float32) + bac_ref[...],
        0.0,
    )
    out = h + out
    logits = jnp.dot(out, wf_ref[...], preferred_element_type=jnp.float32) + bf_ref[...]
    m = jnp.max(logits, axis=-1, keepdims=True)
    lse = jnp.log(jnp.sum(jnp.exp(logits - m), axis=-1, keepdims=True)) + m
    out_ref[...] = logits - lse


def _tc_final(g, ge0, ge1, ge2, ge3, W_ac, b_ac, W_f_pad, b_f_pad):
    full = lambda: pl.BlockSpec((H, H), lambda: (0, 0))
    vec = lambda: pl.BlockSpec((1, H), lambda: (0, 0))
    return pl.pallas_call(
        _tc_final_body,
        grid=(),
        in_specs=[vec(), vec(), vec(), vec(), vec(), full(), vec(), full(), vec()],
        out_specs=pl.BlockSpec((1, H), lambda: (0, 0)),
        out_shape=jax.ShapeDtypeStruct((1, H), jnp.float32),
    )(g, ge0, ge1, ge2, ge3, W_ac, b_ac, W_f_pad, b_f_pad)


def kernel(x, edge_index, W_np, b_np, W_np1, b_np1, W_np2, b_np2, W_init, b_init,
           Wm, bm, wi, bi, wj, bj, gamma, beta, We, be, We1, be1, We2, be2,
           W_ac, b_ac, W_f, b_f):
    r = lambda v: v.reshape(1, -1)

    src = edge_index[0]
    dst = edge_index[1]
    pad = EPAD - E
    arp = jnp.arange(pad, dtype=jnp.int32)
    trash = N + arp % (AGG_ROWS - N)
    src_p = jnp.concatenate([src, arp % N]).reshape(NW, RPT, CH)
    dst_p = jnp.concatenate([dst, trash]).reshape(NW, RPT, CH)

    u, g = _tc_init(x, W_init, r(b_init), W_np, r(b_np), W_np1, r(b_np1),
                    W_np2, r(b_np2))

    s = jnp.ones((1, H), jnp.float32)
    t = jnp.zeros((1, H), jnp.float32)
    ges = []
    for i in range(L):
        um = _tc_pre(u, s, t, Wm[i], r(bm[i]))
        parts = _sc_scatter(um, src_p, dst_p)
        u, s, t, ge = _tc_post(
            parts, um, r(wi[i]), r(bi[i]), r(wj[i]), r(bj[i]),
            r(gamma[(i + 1) % L]), r(beta[(i + 1) % L]),
            We[i], r(be[i]), We1[i], r(be1[i]), We2[i], r(be2[i]))
        ges.append(ge)

    W_f_pad = jnp.zeros((H, H), jnp.float32).at[:, :C].set(W_f)
    b_f_pad = jnp.full((1, H), -1e30, jnp.float32).at[:, :C].set(b_f)
    out = _tc_final(g, ges[0], ges[1], ges[2], ges[3], W_ac, r(b_ac), W_f_pad, b_f_pad)
    return out[:, :C]


# speedup vs baseline: 3.9413x; 1.0536x over previous
import functools

import jax
import jax.numpy as jnp
from jax import lax
from jax.experimental import pallas as pl
from jax.experimental.pallas import tpu as pltpu
from jax.experimental.pallas import tpu_sc as plsc

N = 10000
E = 320000
H = 128
C = 10
L = 4

NC = 2
NS = 16
NW = NC * NS
CH = 128
ROWD = 2
IDXD = 4
EPAD = ((E + NW * CH * IDXD - 1) // (NW * CH * IDXD)) * (NW * CH * IDXD)
RPT = EPAD // (NW * CH)
ZB = 640
AGG_ROWS = NS * ZB

BLK = 1000
INV_AVG = float(N) / float(E)


def _sc_scatter_body(um_hbm, src_hbm, dst_hbm, out_hbm, src_r, dst_r,
                     rows_a, rows_b, agg_s, sem_g, sem_i):
    cid = lax.axis_index("c")
    sid = lax.axis_index("s")
    wid = cid * NS + sid
    bufs = (rows_a, rows_b)

    def idx_load(j, slot):
        return (pltpu.make_async_copy(src_hbm.at[wid].at[j], src_r.at[slot], sem_i),
                pltpu.make_async_copy(dst_hbm.at[wid].at[j], dst_r.at[slot], sem_i))

    def gather(slot, b):
        return pltpu.make_async_copy(um_hbm.at[src_r.at[slot]], bufs[b], sem_g)

    def _zero_row(i, _):
        z = jnp.zeros((16,), jnp.float32)
        for j in range(H // 16):
            rows_a[i, pl.ds(j * 16, 16)] = z
        return 0

    lax.fori_loop(0, CH, _zero_row, 0)
    for k in range(ZB // CH):
        pltpu.sync_copy(rows_a, agg_s.at[pl.ds(sid * ZB + k * CH, CH)])
    plsc.subcore_barrier()

    for s in range(IDXD):
        d1, d2 = idx_load(s, s)
        d1.start(); d2.start()
    for s in range(ROWD):
        d1, d2 = idx_load(s, s)
        d1.wait(); d2.wait()
        gather(s, s).start()

    def _outer(g, _):
        j0 = g * IDXD
        for b in range(IDXD):
            j = j0 + b
            gather(b, b % ROWD).wait()
            pltpu.sync_copy(bufs[b % ROWD], agg_s.at[dst_r.at[b]], add=True)
            d1, d2 = idx_load(j + IDXD, b)
            d1.start(); d2.start()
            nslot = (b + ROWD) % IDXD
            w1, w2 = idx_load(j + ROWD, nslot)
            w1.wait(); w2.wait()
            gather(nslot, b % ROWD).start()
        return 0

    lax.fori_loop(0, RPT // IDXD - 1, _outer, 0)
    j0 = RPT - IDXD
    for b in range(IDXD):
        j = j0 + b
        gather(b, b % ROWD).wait()
        pltpu.sync_copy(bufs[b % ROWD], agg_s.at[dst_r.at[b]], add=True)
        if b + ROWD < IDXD:
            nslot = b + ROWD
            w1, w2 = idx_load(j + ROWD, nslot)
            w1.wait(); w2.wait()
            gather(nslot, b % ROWD).start()
    plsc.subcore_barrier()

    for k in range(ZB // CH):
        sl = pl.ds(sid * ZB + k * CH, CH)
        pltpu.sync_copy(agg_s.at[sl], rows_a)
        pltpu.sync_copy(rows_a, out_hbm.at[cid].at[sl])


@functools.cache
def _sc_scatter_build():
    return pl.kernel(
        _sc_scatter_body,
        out_type=jax.ShapeDtypeStruct((NC, AGG_ROWS, H), jnp.float32),
        mesh=plsc.VectorSubcoreMesh(core_axis_name="c", subcore_axis_name="s",
                                    num_cores=NC, num_subcores=NS),
        scratch_types=[
            pltpu.VMEM((IDXD, CH), jnp.int32),
            pltpu.VMEM((IDXD, CH), jnp.int32),
            pltpu.VMEM((CH, H), jnp.float32),
            pltpu.VMEM((CH, H), jnp.float32),
            pltpu.VMEM_SHARED((AGG_ROWS, H), jnp.float32),
            pltpu.SemaphoreType.DMA,
            pltpu.SemaphoreType.DMA,
        ],
    )


def _sc_scatter(um, src_p, dst_p):
    return _sc_scatter_build()(um, src_p, dst_p)


def _mm(a, b):
    return jnp.dot(a, b, preferred_element_type=jnp.float32)


def _extract(m, we, be, we1, be1, we2, be2):
    ge = _mm(m, we) + be
    h = jnp.maximum(_mm(ge, we1) + be1, 0.0)
    return ge + _mm(h, we2) + be2


def _tc_head_body(x_ref, wi_ref, bi_ref, wm_ref, bm_ref, wn_ref, bn_ref,
                  wn1_ref, bn1_ref, wn2_ref, bn2_ref, um_ref, g_ref, acc_ref):
    i = pl.program_id(0)

    @pl.when(i == 0)
    def _():
        acc_ref[...] = jnp.zeros_like(acc_ref)

    xb = x_ref[...]
    u0 = _mm(xb, wi_ref[...]) + bi_ref[...]
    um_ref[...] = _mm(u0, wm_ref[...]) + bm_ref[...]
    acc_ref[...] += jnp.sum(xb, axis=0, keepdims=True)

    @pl.when(i == pl.num_programs(0) - 1)
    def _():
        m = acc_ref[...] * (1.0 / N)
        g_ref[...] = _extract(m, wn_ref[...], bn_ref[...], wn1_ref[...],
                              bn1_ref[...], wn2_ref[...], bn2_ref[...])


def _tc_head(x, W_init, b_init, Wm0, bm0, W_np, b_np, W_np1, b_np1, W_np2, b_np2):
    full = lambda: pl.BlockSpec((H, H), lambda i: (0, 0))
    vec = lambda: pl.BlockSpec((1, H), lambda i: (0, 0))
    return pl.pallas_call(
        _tc_head_body,
        grid=(N // BLK,),
        in_specs=[
            pl.BlockSpec((BLK, H), lambda i: (i, 0)),
            full(), vec(), full(), vec(),
            full(), vec(), full(), vec(), full(), vec(),
        ],
        out_specs=[
            pl.BlockSpec((BLK, H), lambda i: (i, 0)),
            pl.BlockSpec((1, H), lambda i: (0, 0)),
        ],
        out_shape=[
            jax.ShapeDtypeStruct((N, H), jnp.float32),
            jax.ShapeDtypeStruct((1, H), jnp.float32),
        ],
        scratch_shapes=[pltpu.VMEM((1, H), jnp.float32)],
    )(x, W_init, b_init, Wm0, bm0, W_np, b_np, W_np1, b_np1, W_np2, b_np2)


def _tc_mid_body(p0_ref, p1_ref, um_ref, wi_ref, bi_ref, wj_ref, bj_ref,
                 gam_ref, bet_ref, we_ref, be_ref, we1_ref, be1_ref,
                 we2_ref, be2_ref, wmn_ref, bmn_ref,
                 umn_ref, ge_ref, u_all, accs, accq, s_sc, t_sc):
    p = pl.program_id(0)
    i = pl.program_id(1)

    @pl.when(jnp.logical_and(p == 0, i == 0))
    def _():
        accs[...] = jnp.zeros_like(accs)
        accq[...] = jnp.zeros_like(accq)

    @pl.when(p == 0)
    def _():
        agg = (p0_ref[0] + p1_ref[0]) * INV_AVG
        um = um_ref[...]
        ai = wi_ref[...] * um + bi_ref[...]
        aj = wj_ref[...] * agg + bj_ref[...]
        u = agg + um + ai * aj
        u_all[pl.ds(i * BLK, BLK), :] = u
        accs[...] += jnp.sum(u, axis=0, keepdims=True)
        accq[...] += jnp.sum(u * u, axis=0, keepdims=True)

    @pl.when(jnp.logical_and(p == 0, i == pl.num_programs(1) - 1))
    def _():
        mu = accs[...] * (1.0 / N)
        var = accq[...] * (1.0 / N) - mu * mu
        s = gam_ref[...] * lax.rsqrt(var + 1e-5)
        s_sc[...] = s
        t_sc[...] = bet_ref[...] - mu * s
        ge_ref[...] = _extract(mu, we_ref[...], be_ref[...], we1_ref[...],
                               be1_ref[...], we2_ref[...], be2_ref[...])

    @pl.when(p == 1)
    def _():
        u = u_all[pl.ds(i * BLK, BLK), :]
        un = u * s_sc[...] + t_sc[...]
        umn_ref[...] = _mm(un, wmn_ref[...]) + bmn_ref[...]


def _tc_mid(parts, um, wi_i, bi_i, wj_i, bj_i, gam_n, bet_n,
            We_i, be_i, We1_i, be1_i, We2_i, be2_i, Wm_n, bm_n):
    blk_a = lambda c: pl.BlockSpec(
        (1, BLK, H), lambda p, i: (c, jnp.where(p == 0, i, 0), 0))
    row_a = lambda: pl.BlockSpec((BLK, H), lambda p, i: (jnp.where(p == 0, i, 0), 0))
    full = lambda: pl.BlockSpec((H, H), lambda p, i: (0, 0))
    vec = lambda: pl.BlockSpec((1, H), lambda p, i: (0, 0))
    return pl.pallas_call(
        _tc_mid_body,
        grid=(2, N // BLK),
        in_specs=[
            blk_a(0), blk_a(1), row_a(),
            vec(), vec(), vec(), vec(), vec(), vec(),
            full(), vec(), full(), vec(), full(), vec(),
            full(), vec(),
        ],
        out_specs=[
            pl.BlockSpec((BLK, H), lambda p, i: (jnp.where(p == 1, i, 0), 0)),
            pl.BlockSpec((1, H), lambda p, i: (0, 0)),
        ],
        out_shape=[
            jax.ShapeDtypeStruct((N, H), jnp.float32),
            jax.ShapeDtypeStruct((1, H), jnp.float32),
        ],
        scratch_shapes=[
            pltpu.VMEM((N, H), jnp.float32),
            pltpu.VMEM((1, H), jnp.float32),
            pltpu.VMEM((1, H), jnp.float32),
            pltpu.VMEM((1, H), jnp.float32),
            pltpu.VMEM((1, H), jnp.float32),
        ],
    )(parts, parts, um, wi_i, bi_i, wj_i, bj_i, gam_n, bet_n,
      We_i, be_i, We1_i, be1_i, We2_i, be2_i, Wm_n, bm_n)


def _tc_tail_body(p0_ref, p1_ref, um_ref, wi_ref, bi_ref, wj_ref, bj_ref,
                  we_ref, be_ref, we1_ref, be1_ref, we2_ref, be2_ref,
                  g_ref, ges_ref, wac_ref, bac_ref, wf_ref, bf_ref,
                  out_ref, acc_ref):
    i = pl.program_id(0)

    @pl.when(i == 0)
    def _():
        acc_ref[...] = jnp.zeros_like(acc_ref)

    agg = (p0_ref[0] + p1_ref[0]) * INV_AVG
    um = um_ref[...]
    ai = wi_ref[...] * um + bi_ref[...]
    aj = wj_ref[...] * agg + bj_ref[...]
    u = agg + um + ai * aj
    acc_ref[...] += jnp.sum(u, axis=0, keepdims=True)

    @pl.when(i == pl.num_programs(0) - 1)
    def _():
        mu = acc_ref[...] * (1.0 / N)
        ge3 = _extract(mu, we_ref[...], be_ref[...], we1_ref[...],
                       be1_ref[...], we2_ref[...], be2_ref[...])
        out = g_ref[...] + (ges_ref[...] + ge3) * (1.0 / L)
        h = jnp.maximum(_mm(out, wac_ref[...]) + bac_ref[...], 0.0)
        out = h + out
        logits = _mm(out, wf_ref[...]) + bf_ref[...]
        m = jnp.max(logits, axis=-1, keepdims=True)
        lse = jnp.log(jnp.sum(jnp.exp(logits - m), axis=-1, keepdims=True)) + m
        out_ref[...] = logits - lse


def _tc_tail(parts, um, wi_i, bi_i, wj_i, bj_i,
             We_i, be_i, We1_i, be1_i, We2_i, be2_i,
             g, ges, W_ac, b_ac, W_f_pad, b_f_pad):
    blk_a = lambda c: pl.BlockSpec((1, BLK, H), lambda i: (c, i, 0))
    full = lambda: pl.BlockSpec((H, H), lambda i: (0, 0))
    vec = lambda: pl.BlockSpec((1, H), lambda i: (0, 0))
    return pl.pallas_call(
        _tc_tail_body,
        grid=(N // BLK,),
        in_specs=[
            blk_a(0), blk_a(1),
            pl.BlockSpec((BLK, H), lambda i: (i, 0)),
            vec(), vec(), vec(), vec(),
            full(), vec(), full(), vec(), full(), vec(),
            vec(), vec(), full(), vec(), full(), vec(),
        ],
        out_specs=pl.BlockSpec((1, H), lambda i: (0, 0)),
        out_shape=jax.ShapeDtypeStruct((1, H), jnp.float32),
        scratch_shapes=[pltpu.VMEM((1, H), jnp.float32)],
    )(parts, parts, um, wi_i, bi_i, wj_i, bj_i,
      We_i, be_i, We1_i, be1_i, We2_i, be2_i,
      g, ges, W_ac, b_ac, W_f_pad, b_f_pad)


def kernel(x, edge_index, W_np, b_np, W_np1, b_np1, W_np2, b_np2, W_init, b_init,
           Wm, bm, wi, bi, wj, bj, gamma, beta, We, be, We1, be1, We2, be2,
           W_ac, b_ac, W_f, b_f):
    r = lambda v: v.reshape(1, -1)

    src = edge_index[0]
    dst = edge_index[1]
    pad = EPAD - E
    arp = jnp.arange(pad, dtype=jnp.int32)
    trash = N + arp % (AGG_ROWS - N)
    src_p = jnp.concatenate([src, arp % N]).reshape(NW, RPT, CH)
    dst_p = jnp.concatenate([dst, trash]).reshape(NW, RPT, CH)

    um, g = _tc_head(x, W_init, r(b_init), Wm[0], r(bm[0]),
                     W_np, r(b_np), W_np1, r(b_np1), W_np2, r(b_np2))

    ges = jnp.zeros((1, H), jnp.float32)
    for i in range(L - 1):
        parts = _sc_scatter(um, src_p, dst_p)
        um, ge = _tc_mid(
            parts, um, r(wi[i]), r(bi[i]), r(wj[i]), r(bj[i]),
            r(gamma[i + 1]), r(beta[i + 1]),
            We[i], r(be[i]), We1[i], r(be1[i]), We2[i], r(be2[i]),
            Wm[i + 1], r(bm[i + 1]))
        ges = ges + ge

    parts = _sc_scatter(um, src_p, dst_p)
    W_f_pad = jnp.zeros((H, H), jnp.float32).at[:, :C].set(W_f)
    b_f_pad = jnp.full((1, H), -1e30, jnp.float32).at[:, :C].set(b_f)
    out = _tc_tail(parts, um, r(wi[3]), r(bi[3]), r(wj[3]), r(bj[3]),
                   We[3], r(be[3]), We1[3], r(be1[3]), We2[3], r(be2[3]),
                   g, ges, W_ac, r(b_ac), W_f_pad, b_f_pad)
    return out[:, :C]


# speedup vs baseline: 4.0125x; 1.0181x over previous
import functools

import jax
import jax.numpy as jnp
from jax import lax
from jax.experimental import pallas as pl
from jax.experimental.pallas import tpu as pltpu
from jax.experimental.pallas import tpu_sc as plsc

N = 10000
E = 320000
H = 128
C = 10
L = 4

NC = 2
NS = 16
NW = NC * NS
CH = 128
ROWD = 2
IDXD = 4
EPAD = ((E + NW * CH * IDXD - 1) // (NW * CH * IDXD)) * (NW * CH * IDXD)
RPT = EPAD // (NW * CH)
ZB = 640
AGG_ROWS = NS * ZB

BLK = 1000
INV_AVG = float(N) / float(E)


def _sc_scatter_body(um_hbm, src_hbm, dst_hbm, out_hbm, src_r, dst_r,
                     rows_a, rows_b, agg_s, sem_g, sem_i):
    cid = lax.axis_index("c")
    sid = lax.axis_index("s")
    wid = cid * NS + sid
    bufs = (rows_a, rows_b)

    def idx_load(j, slot):
        return (pltpu.make_async_copy(src_hbm.at[wid].at[j], src_r.at[slot], sem_i),
                pltpu.make_async_copy(dst_hbm.at[wid].at[j], dst_r.at[slot], sem_i))

    def gather(slot, b):
        return pltpu.make_async_copy(um_hbm.at[src_r.at[slot]], bufs[b], sem_g)

    for s in range(IDXD):
        d1, d2 = idx_load(s, s)
        d1.start(); d2.start()

    def _zero_row(i, _):
        z = jnp.zeros((16,), jnp.float32)
        for j in range(H // 16):
            rows_a[i, pl.ds(j * 16, 16)] = z
        return 0

    lax.fori_loop(0, CH, _zero_row, 0)
    for k in range(ZB // CH):
        pltpu.sync_copy(rows_a, agg_s.at[pl.ds(sid * ZB + k * CH, CH)])
    plsc.subcore_barrier()

    for s in range(ROWD):
        d1, d2 = idx_load(s, s)
        d1.wait(); d2.wait()
        gather(s, s).start()

    def _outer(g, _):
        j0 = g * IDXD
        for b in range(IDXD):
            j = j0 + b
            gather(b, b % ROWD).wait()
            pltpu.sync_copy(bufs[b % ROWD], agg_s.at[dst_r.at[b]], add=True)
            d1, d2 = idx_load(j + IDXD, b)
            d1.start(); d2.start()
            nslot = (b + ROWD) % IDXD
            w1, w2 = idx_load(j + ROWD, nslot)
            w1.wait(); w2.wait()
            gather(nslot, b % ROWD).start()
        return 0

    lax.fori_loop(0, RPT // IDXD - 1, _outer, 0)
    j0 = RPT - IDXD
    for b in range(IDXD):
        j = j0 + b
        gather(b, b % ROWD).wait()
        pltpu.sync_copy(bufs[b % ROWD], agg_s.at[dst_r.at[b]], add=True)
        if b + ROWD < IDXD:
            nslot = b + ROWD
            w1, w2 = idx_load(j + ROWD, nslot)
            w1.wait(); w2.wait()
            gather(nslot, b % ROWD).start()
    plsc.subcore_barrier()

    sl = lambda k: pl.ds(sid * ZB + k * CH, CH)
    nwb = ZB // CH
    pltpu.sync_copy(agg_s.at[sl(0)], rows_a)
    writes = [None] * nwb
    for k in range(nwb):
        w = pltpu.make_async_copy(bufs[k % ROWD], out_hbm.at[cid].at[sl(k)], sem_g)
        w.start()
        writes[k] = w
        if k + 1 < nwb:
            if k >= 1:
                writes[k - 1].wait()
            pltpu.sync_copy(agg_s.at[sl(k + 1)], bufs[(k + 1) % ROWD])
    writes[nwb - 2].wait()
    writes[nwb - 1].wait()


@functools.cache
def _sc_scatter_build():
    return pl.kernel(
        _sc_scatter_body,
        out_type=jax.ShapeDtypeStruct((NC, AGG_ROWS, H), jnp.float32),
        mesh=plsc.VectorSubcoreMesh(core_axis_name="c", subcore_axis_name="s",
                                    num_cores=NC, num_subcores=NS),
        scratch_types=[
            pltpu.VMEM((IDXD, CH), jnp.int32),
            pltpu.VMEM((IDXD, CH), jnp.int32),
            pltpu.VMEM((CH, H), jnp.float32),
            pltpu.VMEM((CH, H), jnp.float32),
            pltpu.VMEM_SHARED((AGG_ROWS, H), jnp.float32),
            pltpu.SemaphoreType.DMA,
            pltpu.SemaphoreType.DMA,
        ],
    )


def _sc_scatter(um, src_p, dst_p):
    return _sc_scatter_build()(um, src_p, dst_p)


def _mm(a, b):
    return jnp.dot(a, b, preferred_element_type=jnp.float32)


def _extract(m, we, be, we1, be1, we2, be2):
    ge = _mm(m, we) + be
    h = jnp.maximum(_mm(ge, we1) + be1, 0.0)
    return ge + _mm(h, we2) + be2


def _tc_head_body(x_ref, wi_ref, bi_ref, wm_ref, bm_ref, wn_ref, bn_ref,
                  wn1_ref, bn1_ref, wn2_ref, bn2_ref, um_ref, g_ref, acc_ref):
    i = pl.program_id(0)

    @pl.when(i == 0)
    def _():
        acc_ref[...] = jnp.zeros_like(acc_ref)

    xb = x_ref[...]
    u0 = _mm(xb, wi_ref[...]) + bi_ref[...]
    um_ref[...] = _mm(u0, wm_ref[...]) + bm_ref[...]
    acc_ref[...] += jnp.sum(xb, axis=0, keepdims=True)

    @pl.when(i == pl.num_programs(0) - 1)
    def _():
        m = acc_ref[...] * (1.0 / N)
        g_ref[...] = _extract(m, wn_ref[...], bn_ref[...], wn1_ref[...],
                              bn1_ref[...], wn2_ref[...], bn2_ref[...])


def _tc_head(x, W_init, b_init, Wm0, bm0, W_np, b_np, W_np1, b_np1, W_np2, b_np2):
    full = lambda: pl.BlockSpec((H, H), lambda i: (0, 0))
    vec = lambda: pl.BlockSpec((1, H), lambda i: (0, 0))
    return pl.pallas_call(
        _tc_head_body,
        grid=(N // BLK,),
        in_specs=[
            pl.BlockSpec((BLK, H), lambda i: (i, 0)),
            full(), vec(), full(), vec(),
            full(), vec(), full(), vec(), full(), vec(),
        ],
        out_specs=[
            pl.BlockSpec((BLK, H), lambda i: (i, 0)),
            pl.BlockSpec((1, H), lambda i: (0, 0)),
        ],
        out_shape=[
            jax.ShapeDtypeStruct((N, H), jnp.float32),
            jax.ShapeDtypeStruct((1, H), jnp.float32),
        ],
        scratch_shapes=[pltpu.VMEM((1, H), jnp.float32)],
    )(x, W_init, b_init, Wm0, bm0, W_np, b_np, W_np1, b_np1, W_np2, b_np2)


def _tc_mid_body(p0_ref, p1_ref, um_ref, wi_ref, bi_ref, wj_ref, bj_ref,
                 gam_ref, bet_ref, we_ref, be_ref, we1_ref, be1_ref,
                 we2_ref, be2_ref, wmn_ref, bmn_ref,
                 umn_ref, ge_ref, u_all, accs, accq, s_sc, t_sc):
    p = pl.program_id(0)
    i = pl.program_id(1)

    @pl.when(jnp.logical_and(p == 0, i == 0))
    def _():
        accs[...] = jnp.zeros_like(accs)
        accq[...] = jnp.zeros_like(accq)

    @pl.when(p == 0)
    def _():
        agg = (p0_ref[0] + p1_ref[0]) * INV_AVG
        um = um_ref[...]
        ai = wi_ref[...] * um + bi_ref[...]
        aj = wj_ref[...] * agg + bj_ref[...]
        u = agg + um + ai * aj
        u_all[pl.ds(i * BLK, BLK), :] = u
        accs[...] += jnp.sum(u, axis=0, keepdims=True)
        accq[...] += jnp.sum(u * u, axis=0, keepdims=True)

    @pl.when(jnp.logical_and(p == 0, i == pl.num_programs(1) - 1))
    def _():
        mu = accs[...] * (1.0 / N)
        var = accq[...] * (1.0 / N) - mu * mu
        s = gam_ref[...] * lax.rsqrt(var + 1e-5)
        s_sc[...] = s
        t_sc[...] = bet_ref[...] - mu * s
        ge_ref[...] = _extract(mu, we_ref[...], be_ref[...], we1_ref[...],
                               be1_ref[...], we2_ref[...], be2_ref[...])

    @pl.when(p == 1)
    def _():
        u = u_all[pl.ds(i * BLK, BLK), :]
        un = u * s_sc[...] + t_sc[...]
        umn_ref[...] = _mm(un, wmn_ref[...]) + bmn_ref[...]


def _tc_mid(parts, um, wi_i, bi_i, wj_i, bj_i, gam_n, bet_n,
            We_i, be_i, We1_i, be1_i, We2_i, be2_i, Wm_n, bm_n):
    blk_a = lambda c: pl.BlockSpec(
        (1, BLK, H), lambda p, i: (c, jnp.where(p == 0, i, 0), 0))
    row_a = lambda: pl.BlockSpec((BLK, H), lambda p, i: (jnp.where(p == 0, i, 0), 0))
    full = lambda: pl.BlockSpec((H, H), lambda p, i: (0, 0))
    vec = lambda: pl.BlockSpec((1, H), lambda p, i: (0, 0))
    return pl.pallas_call(
        _tc_mid_body,
        grid=(2, N // BLK),
        in_specs=[
            blk_a(0), blk_a(1), row_a(),
            vec(), vec(), vec(), vec(), vec(), vec(),
            full(), vec(), full(), vec(), full(), vec(),
            full(), vec(),
        ],
        out_specs=[
            pl.BlockSpec((BLK, H), lambda p, i: (jnp.where(p == 1, i, 0), 0)),
            pl.BlockSpec((1, H), lambda p, i: (0, 0)),
        ],
        out_shape=[
            jax.ShapeDtypeStruct((N, H), jnp.float32),
            jax.ShapeDtypeStruct((1, H), jnp.float32),
        ],
        scratch_shapes=[
            pltpu.VMEM((N, H), jnp.float32),
            pltpu.VMEM((1, H), jnp.float32),
            pltpu.VMEM((1, H), jnp.float32),
            pltpu.VMEM((1, H), jnp.float32),
            pltpu.VMEM((1, H), jnp.float32),
        ],
    )(parts, parts, um, wi_i, bi_i, wj_i, bj_i, gam_n, bet_n,
      We_i, be_i, We1_i, be1_i, We2_i, be2_i, Wm_n, bm_n)


def _tc_tail_body(p0_ref, p1_ref, um_ref, wi_ref, bi_ref, wj_ref, bj_ref,
                  we_ref, be_ref, we1_ref, be1_ref, we2_ref, be2_ref,
                  g_ref, ges_ref, wac_ref, bac_ref, wf_ref, bf_ref,
                  out_ref, acc_ref):
    i = pl.program_id(0)

    @pl.when(i == 0)
    def _():
        acc_ref[...] = jnp.zeros_like(acc_ref)

    agg = (p0_ref[0] + p1_ref[0]) * INV_AVG
    um = um_ref[...]
    ai = wi_ref[...] * um + bi_ref[...]
    aj = wj_ref[...] * agg + bj_ref[...]
    u = agg + um + ai * aj
    acc_ref[...] += jnp.sum(u, axis=0, keepdims=True)

    @pl.when(i == pl.num_programs(0) - 1)
    def _():
        mu = acc_ref[...] * (1.0 / N)
        ge3 = _extract(mu, we_ref[...], be_ref[...], we1_ref[...],
                       be1_ref[...], we2_ref[...], be2_ref[...])
        out = g_ref[...] + (ges_ref[...] + ge3) * (1.0 / L)
        h = jnp.maximum(_mm(out, wac_ref[...]) + bac_ref[...], 0.0)
        out = h + out
        logits = _mm(out, wf_ref[...]) + bf_ref[...]
        m = jnp.max(logits, axis=-1, keepdims=True)
        lse = jnp.log(jnp.sum(jnp.exp(logits - m), axis=-1, keepdims=True)) + m
        out_ref[...] = logits - lse


def _tc_tail(parts, um, wi_i, bi_i, wj_i, bj_i,
             We_i, be_i, We1_i, be1_i, We2_i, be2_i,
             g, ges, W_ac, b_ac, W_f_pad, b_f_pad):
    blk_a = lambda c: pl.BlockSpec((1, BLK, H), lambda i: (c, i, 0))
    full = lambda: pl.BlockSpec((H, H), lambda i: (0, 0))
    vec = lambda: pl.BlockSpec((1, H), lambda i: (0, 0))
    return pl.pallas_call(
        _tc_tail_body,
        grid=(N // BLK,),
        in_specs=[
            blk_a(0), blk_a(1),
            pl.BlockSpec((BLK, H), lambda i: (i, 0)),
            vec(), vec(), vec(), vec(),
            full(), vec(), full(), vec(), full(), vec(),
            vec(), vec(), full(), vec(), full(), vec(),
        ],
        out_specs=pl.BlockSpec((1, H), lambda i: (0, 0)),
        out_shape=jax.ShapeDtypeStruct((1, H), jnp.float32),
        scratch_shapes=[pltpu.VMEM((1, H), jnp.float32)],
    )(parts, parts, um, wi_i, bi_i, wj_i, bj_i,
      We_i, be_i, We1_i, be1_i, We2_i, be2_i,
      g, ges, W_ac, b_ac, W_f_pad, b_f_pad)


def kernel(x, edge_index, W_np, b_np, W_np1, b_np1, W_np2, b_np2, W_init, b_init,
           Wm, bm, wi, bi, wj, bj, gamma, beta, We, be, We1, be1, We2, be2,
           W_ac, b_ac, W_f, b_f):
    r = lambda v: v.reshape(1, -1)

    src = edge_index[0]
    dst = edge_index[1]
    pad = EPAD - E
    arp = jnp.arange(pad, dtype=jnp.int32)
    trash = N + arp % (AGG_ROWS - N)
    src_p = jnp.concatenate([src, arp % N]).reshape(NW, RPT, CH)
    dst_p = jnp.concatenate([dst, trash]).reshape(NW, RPT, CH)

    um, g = _tc_head(x, W_init, r(b_init), Wm[0], r(bm[0]),
                     W_np, r(b_np), W_np1, r(b_np1), W_np2, r(b_np2))

    ges = jnp.zeros((1, H), jnp.float32)
    for i in range(L - 1):
        parts = _sc_scatter(um, src_p, dst_p)
        um, ge = _tc_mid(
            parts, um, r(wi[i]), r(bi[i]), r(wj[i]), r(bj[i]),
            r(gamma[i + 1]), r(beta[i + 1]),
            We[i], r(be[i]), We1[i], r(be1[i]), We2[i], r(be2[i]),
            Wm[i + 1], r(bm[i + 1]))
        ges = ges + ge

    parts = _sc_scatter(um, src_p, dst_p)
    W_f_pad = jnp.zeros((H, H), jnp.float32).at[:, :C].set(W_f)
    b_f_pad = jnp.full((1, H), -1e30, jnp.float32).at[:, :C].set(b_f)
    out = _tc_tail(parts, um, r(wi[3]), r(bi[3]), r(wj[3]), r(bj[3]),
                   We[3], r(be[3]), We1[3], r(be1[3]), We2[3], r(be2[3]),
                   g, ges, W_ac, r(b_ac), W_f_pad, b_f_pad)
    return out[:, :C]
